# baseline (speedup 1.0000x reference)
import math

import jax
import jax.numpy as jnp
from jax.experimental import pallas as pl

S = 64.0
M2 = 0.5
COS_M = math.cos(M2)
SIN_M = math.sin(M2)
THETA = math.cos(math.pi - M2)
SINMM = math.sin(math.pi - M2) * M2

B = 1024
C = 100000
BC = 2048


def _margin(t):
    sin_theta = jnp.sqrt(jnp.maximum(1.0 - t * t, 0.0))
    cos_theta_m = t * COS_M - sin_theta * SIN_M
    return jnp.where(t > THETA, cos_theta_m, t - SINMM)


def _body(labels_ref, logits_ref, out_ref):
    j = pl.program_id(0)
    x = logits_ref[...]
    cols = j * BC + jax.lax.broadcasted_iota(jnp.int32, x.shape, 1)
    mask = cols == labels_ref[...]
    out_ref[...] = jnp.where(mask, _margin(x), x) * S


def kernel(logits, norms, labels):
    del norms
    labels2d = labels.reshape(B, 1)
    grid = (pl.cdiv(C, BC),)
    return pl.pallas_call(
        _body,
        grid=grid,
        in_specs=[
            pl.BlockSpec((B, 1), lambda j: (0, 0)),
            pl.BlockSpec((B, BC), lambda j: (0, j)),
        ],
        out_specs=pl.BlockSpec((B, BC), lambda j: (0, j)),
        out_shape=jax.ShapeDtypeStruct((B, C), jnp.float32),
    )(labels2d, logits)
